# baseline (speedup 1.0000x reference)
import jax
import jax.numpy as jnp
from jax.experimental import pallas as pl
from jax.experimental.pallas import tpu as pltpu

POOL = 64
D_MODEL = 768
D_FF = 32
S = 2048
K = 2
COLS = POOL * D_FF

TILE = 512


def _kernel(x_ref, idx_ref, w_in_ref, w_out_ref, b_in_ref, b_out_ref,
            out_ref, w2_ref):
    i = pl.program_id(0)

    @pl.when(i == 0)
    def _build_w2():
        w = jnp.transpose(w_in_ref[...], (0, 2, 1))
        w2_ref[...] = w.reshape(COLS, D_MODEL).astype(jnp.bfloat16)

    x = x_ref[...].astype(jnp.bfloat16)
    w_out = w_out_ref[...].astype(jnp.bfloat16)
    b_in = b_in_ref[...]

    h = jax.lax.dot_general(
        x, w2_ref[...], (((1,), (1,)), ((), ())),
        preferred_element_type=jnp.float32) + b_in
    g = 0.5 * h * (1.0 + jax.lax.erf(h * 0.7071067811865476))

    col_expert = jax.lax.broadcasted_iota(jnp.int32, (TILE, COLS), 1) // D_FF
    pool_iota = jax.lax.broadcasted_iota(jnp.int32, (TILE, POOL), 1)

    idx = idx_ref[...]
    for k in range(K):
        ik = idx[:, k][:, None]
        gk = jnp.where(ik == col_expert, g, 0.0).astype(jnp.bfloat16)
        ok = jnp.dot(gk, w_out, preferred_element_type=jnp.float32)
        onehot = (ik == pool_iota).astype(jnp.float32)
        ok = ok + jnp.dot(onehot, b_out_ref[...],
                          preferred_element_type=jnp.float32)
        out_ref[:, k * D_MODEL:(k + 1) * D_MODEL] = ok


def kernel(x, indices, w_in, w_out, bias_in, bias_out):
    B = x.shape[0]
    x2 = x.reshape(B * S, D_MODEL)
    idx2 = indices.reshape(B * S, K)
    w_out2 = w_out.reshape(COLS, D_MODEL)
    b_in2 = bias_in.reshape(1, COLS)

    n_tiles = (B * S) // TILE
    out = pl.pallas_call(
        _kernel,
        grid=(n_tiles,),
        in_specs=[
            pl.BlockSpec((TILE, D_MODEL), lambda i: (i, 0)),
            pl.BlockSpec((TILE, K), lambda i: (i, 0)),
            pl.BlockSpec((POOL, D_MODEL, D_FF), lambda i: (0, 0, 0)),
            pl.BlockSpec((COLS, D_MODEL), lambda i: (0, 0)),
            pl.BlockSpec((1, COLS), lambda i: (0, 0)),
            pl.BlockSpec((POOL, D_MODEL), lambda i: (0, 0)),
        ],
        out_specs=pl.BlockSpec((TILE, K * D_MODEL), lambda i: (i, 0)),
        out_shape=jax.ShapeDtypeStruct((B * S, K * D_MODEL), jnp.float32),
        scratch_shapes=[pltpu.VMEM((COLS, D_MODEL), jnp.bfloat16)],
    )(x2, idx2, w_in, w_out2, b_in2, bias_out)

    return out.reshape(B, S, K, D_MODEL)


# speedup vs baseline: 1.2836x; 1.2836x over previous
import jax
import jax.numpy as jnp
from jax.experimental import pallas as pl

POOL = 64
D_MODEL = 768
D_FF = 32
S = 2048
K = 2
COLS = POOL * D_FF

TILE = 512


def _kernel(x_ref, idx_ref, w_in_ref, w_out_ref, b_in_ref, b_out_ref,
            out_ref):
    x = x_ref[...].astype(jnp.bfloat16)
    w_out = w_out_ref[...].astype(jnp.bfloat16)
    b_in = b_in_ref[...]

    h = jax.lax.dot_general(
        x, w_in_ref[...], (((1,), (1,)), ((), ())),
        preferred_element_type=jnp.float32) + b_in
    g = 0.5 * h * (1.0 + jax.lax.erf(h * 0.7071067811865476))

    col_expert = jax.lax.broadcasted_iota(jnp.int32, (TILE, COLS), 1) // D_FF
    pool_iota = jax.lax.broadcasted_iota(jnp.int32, (TILE, POOL), 1)

    idx = idx_ref[...]
    for k in range(K):
        ik = idx[:, k][:, None]
        gk = jnp.where(ik == col_expert, g, 0.0).astype(jnp.bfloat16)
        ok = jnp.dot(gk, w_out, preferred_element_type=jnp.float32)
        onehot = (ik == pool_iota).astype(jnp.float32)
        ok = ok + jnp.dot(onehot, b_out_ref[...],
                          preferred_element_type=jnp.float32)
        out_ref[:, k * D_MODEL:(k + 1) * D_MODEL] = ok


def kernel(x, indices, w_in, w_out, bias_in, bias_out):
    B = x.shape[0]
    x2 = x.reshape(B * S, D_MODEL)
    idx2 = indices.reshape(B * S, K)
    w_in2 = jnp.transpose(w_in.astype(jnp.bfloat16), (0, 2, 1))
    w_in2 = w_in2.reshape(COLS, D_MODEL)
    w_out2 = w_out.reshape(COLS, D_MODEL)
    b_in2 = bias_in.reshape(1, COLS)

    n_tiles = (B * S) // TILE
    out = pl.pallas_call(
        _kernel,
        grid=(n_tiles,),
        in_specs=[
            pl.BlockSpec((TILE, D_MODEL), lambda i: (i, 0)),
            pl.BlockSpec((TILE, K), lambda i: (i, 0)),
            pl.BlockSpec((COLS, D_MODEL), lambda i: (0, 0)),
            pl.BlockSpec((COLS, D_MODEL), lambda i: (0, 0)),
            pl.BlockSpec((1, COLS), lambda i: (0, 0)),
            pl.BlockSpec((POOL, D_MODEL), lambda i: (0, 0)),
        ],
        out_specs=pl.BlockSpec((TILE, K * D_MODEL), lambda i: (i, 0)),
        out_shape=jax.ShapeDtypeStruct((B * S, K * D_MODEL), jnp.float32),
    )(x2, idx2, w_in2, w_out2, b_in2, bias_out)

    return out.reshape(B, S, K, D_MODEL)


# speedup vs baseline: 2.6371x; 2.0544x over previous
import jax
import jax.numpy as jnp
from jax.experimental import pallas as pl

POOL = 64
D_MODEL = 768
D_FF = 32
S = 2048
K = 2
COLS = POOL * D_FF

TILE = 512


def _kernel(x_ref, idx_ref, w_in_ref, w_out_ref, b_in_ref, b_out_ref,
            out_ref):
    x = x_ref[...].astype(jnp.bfloat16)
    w_in = w_in_ref[...].astype(jnp.bfloat16)
    w_out = w_out_ref[...].astype(jnp.bfloat16)
    b_in = b_in_ref[...]

    h = jax.lax.dot_general(
        x, w_in, (((1,), (1,)), ((), ())),
        preferred_element_type=jnp.float32) + b_in
    g = 0.5 * h * (1.0 + jax.lax.erf(h * 0.7071067811865476))

    col_expert = jax.lax.broadcasted_iota(jnp.int32, (TILE, COLS), 1) // D_FF
    pool_iota = jax.lax.broadcasted_iota(jnp.int32, (TILE, POOL), 1)

    idx = idx_ref[...]
    for k in range(K):
        ik = idx[:, k][:, None]
        gk = jnp.where(ik == col_expert, g, 0.0).astype(jnp.bfloat16)
        ok = jnp.dot(gk, w_out, preferred_element_type=jnp.float32)
        onehot = (ik == pool_iota).astype(jnp.float32)
        ok = ok + jnp.dot(onehot, b_out_ref[...],
                          preferred_element_type=jnp.float32)
        out_ref[:, k, :] = ok


def kernel(x, indices, w_in, w_out, bias_in, bias_out):
    B = x.shape[0]
    x2 = x.reshape(B * S, D_MODEL)
    idx2 = indices.reshape(B * S, K)
    w_in2 = jnp.transpose(w_in, (0, 2, 1)).reshape(COLS, D_MODEL)
    w_out2 = w_out.reshape(COLS, D_MODEL)
    b_in2 = bias_in.reshape(1, COLS)

    n_tiles = (B * S) // TILE
    out = pl.pallas_call(
        _kernel,
        grid=(n_tiles,),
        in_specs=[
            pl.BlockSpec((TILE, D_MODEL), lambda i: (i, 0)),
            pl.BlockSpec((TILE, K), lambda i: (i, 0)),
            pl.BlockSpec((COLS, D_MODEL), lambda i: (0, 0)),
            pl.BlockSpec((COLS, D_MODEL), lambda i: (0, 0)),
            pl.BlockSpec((1, COLS), lambda i: (0, 0)),
            pl.BlockSpec((POOL, D_MODEL), lambda i: (0, 0)),
        ],
        out_specs=pl.BlockSpec((TILE, K, D_MODEL), lambda i: (i, 0, 0)),
        out_shape=jax.ShapeDtypeStruct((B * S, K, D_MODEL), jnp.float32),
    )(x2, idx2, w_in2, w_out2, b_in2, bias_out)

    return out.reshape(B, S, K, D_MODEL)
